# baseline (speedup 1.0000x reference)
import jax
import jax.numpy as jnp
from jax.experimental import pallas as pl
from jax.experimental.pallas import tpu as pltpu


def _colsum_kernel(xt_ref, w_ref, b_ref, o_ref):
    d, tbl = xt_ref.shape
    wb = jnp.broadcast_to(w_ref[...], (d, tbl))
    y = jnp.sum(xt_ref[...] * wb, axis=0, keepdims=True) + b_ref[0, 0]
    o_ref[...] = y.reshape(o_ref.shape).astype(o_ref.dtype)


def kernel(x, weight, bias):
    B, D = x.shape
    dtype = x.dtype
    L = 128
    Bp = ((B + L - 1) // L) * L
    if Bp != B:
        x = jnp.pad(x, ((0, Bp - B), (0, 0)))

    xt = jnp.swapaxes(x, 0, 1)
    w_col = weight.reshape(D, 1).astype(dtype)
    b2 = bias.reshape(1, 1).astype(jnp.float32)

    TBL = min(131072, Bp)
    out = pl.pallas_call(
        _colsum_kernel,
        out_shape=jax.ShapeDtypeStruct((Bp // L, L), dtype),
        grid=(Bp // TBL,),
        in_specs=[
            pl.BlockSpec((D, TBL), lambda i: (0, i)),
            pl.BlockSpec((D, 1), lambda i: (0, 0)),
            pl.BlockSpec(memory_space=pltpu.SMEM),
        ],
        out_specs=pl.BlockSpec((TBL // L, L), lambda i: (i, 0)),
        compiler_params=pltpu.CompilerParams(
            dimension_semantics=("parallel",)),
    )(xt, w_col, b2)

    return out.reshape(Bp, 1)[:B]


# speedup vs baseline: 1.0184x; 1.0184x over previous
import jax
import jax.numpy as jnp
from jax.experimental import pallas as pl
from jax.experimental.pallas import tpu as pltpu


def _colsum_kernel(xt_ref, w_ref, b_ref, o_ref):
    d, tbl = xt_ref.shape
    wb = jnp.broadcast_to(w_ref[...], (d, tbl))
    y = jnp.sum(xt_ref[...] * wb, axis=0, keepdims=True) + b_ref[0, 0]
    o_ref[...] = y.reshape(o_ref.shape).astype(o_ref.dtype)


def kernel(x, weight, bias):
    B, D = x.shape
    dtype = x.dtype
    L = 128
    Bp = ((B + L - 1) // L) * L
    if Bp != B:
        x = jnp.pad(x, ((0, Bp - B), (0, 0)))

    xt = jnp.swapaxes(x, 0, 1)
    w_col = weight.reshape(D, 1).astype(dtype)
    b2 = bias.reshape(1, 1).astype(jnp.float32)

    TBL = min(65536, Bp)
    out = pl.pallas_call(
        _colsum_kernel,
        out_shape=jax.ShapeDtypeStruct((Bp // L, L), dtype),
        grid=(Bp // TBL,),
        in_specs=[
            pl.BlockSpec((D, TBL), lambda i: (0, i)),
            pl.BlockSpec((D, 1), lambda i: (0, 0)),
            pl.BlockSpec(memory_space=pltpu.SMEM),
        ],
        out_specs=pl.BlockSpec((TBL // L, L), lambda i: (i, 0)),
        compiler_params=pltpu.CompilerParams(
            dimension_semantics=("parallel",)),
    )(xt, w_col, b2)

    return out.reshape(Bp, 1)[:B]


# speedup vs baseline: 1.0228x; 1.0044x over previous
import jax
import jax.numpy as jnp
from jax.experimental import pallas as pl
from jax.experimental.pallas import tpu as pltpu


def _colsum_kernel(xt_ref, w_ref, b_ref, o_ref):
    d, tbl = xt_ref.shape
    wb = jnp.broadcast_to(w_ref[...], (d, tbl))
    y = jnp.sum(xt_ref[...] * wb, axis=0, keepdims=True) + b_ref[0, 0]
    o_ref[...] = y.reshape(o_ref.shape).astype(o_ref.dtype)


def kernel(x, weight, bias):
    B, D = x.shape
    dtype = x.dtype
    L = 128
    Bp = ((B + L - 1) // L) * L
    if Bp != B:
        x = jnp.pad(x, ((0, Bp - B), (0, 0)))

    xt = jnp.swapaxes(x, 0, 1)
    w_col = weight.reshape(D, 1).astype(dtype)
    b2 = bias.reshape(1, 1).astype(jnp.float32)

    TBL = min(65536, Bp)
    out = pl.pallas_call(
        _colsum_kernel,
        out_shape=jax.ShapeDtypeStruct((Bp // L, L), dtype),
        grid=(pl.cdiv(Bp, TBL),),
        in_specs=[
            pl.BlockSpec((D, TBL), lambda i: (0, i)),
            pl.BlockSpec((D, 1), lambda i: (0, 0)),
            pl.BlockSpec(memory_space=pltpu.SMEM),
        ],
        out_specs=pl.BlockSpec((TBL // L, L), lambda i: (i, 0)),
        compiler_params=pltpu.CompilerParams(
            dimension_semantics=("parallel",)),
    )(xt, w_col, b2)

    return out.reshape(Bp, 1)[:B]
